# baseline (speedup 1.0000x reference)
import functools

import jax
import jax.numpy as jnp
from jax import lax
from jax.experimental import pallas as pl
from jax.experimental.pallas import tpu as pltpu
from jax.experimental.pallas import tpu_sc as plsc

B, N, D, M = 8, 2048, 256, 18
ROW = N * D
PM_ROW = M * D
TIF_ROW = ROW + PM_ROW
A_ROW = N * 9
TA_ROW = (N + M) * 9
EA_ROW = M * 9

W_PER_B = 4
CH = 32768
NCH = ROW // (W_PER_B * CH)


def _sc_feature_copy(af2, pm2):
    info = plsc.get_sparse_core_info()
    nc = info.num_cores
    mesh = plsc.VectorSubcoreMesh(core_axis_name="c", subcore_axis_name="s")

    @functools.partial(
        pl.kernel,
        mesh=mesh,
        out_type=[
            jax.ShapeDtypeStruct((B, TIF_ROW), jnp.float32),
            jax.ShapeDtypeStruct((B, PM_ROW), jnp.float32),
        ],
        scratch_types=[
            pltpu.VMEM((CH,), jnp.float32),
            pltpu.VMEM((CH,), jnp.float32),
            pltpu.VMEM((PM_ROW,), jnp.float32),
            pltpu.SemaphoreType.DMA,
            pltpu.SemaphoreType.DMA,
            pltpu.SemaphoreType.DMA,
            pltpu.SemaphoreType.DMA,
            pltpu.SemaphoreType.DMA,
        ],
    )
    def k(af_hbm, pm_hbm, tif_hbm, ef_hbm, buf0, buf1, pbuf, si0, si1, so0, so1, sp):
        w = lax.axis_index("s") * nc + lax.axis_index("c")
        b = w // W_PER_B
        q = w % W_PER_B
        base = q * (NCH * CH)
        bufs = (buf0, buf1)
        sin = (si0, si1)
        sout = (so0, so1)

        cin = [None, None]
        for s in range(2):
            cin[s] = pltpu.async_copy(
                af_hbm.at[b, pl.ds(base + s * CH, CH)], bufs[s], sin[s])
        for i in range(NCH):
            s = i % 2
            cin[s].wait()
            cout = pltpu.async_copy(
                bufs[s], tif_hbm.at[b, pl.ds(base + i * CH, CH)], sout[s])
            cout.wait()
            if i + 2 < NCH:
                cin[s] = pltpu.async_copy(
                    af_hbm.at[b, pl.ds(base + (i + 2) * CH, CH)], bufs[s], sin[s])

        @pl.when(q == 0)
        def _pm():
            pltpu.async_copy(pm_hbm.at[b], pbuf, sp).wait()
            pltpu.async_copy(pbuf, tif_hbm.at[b, pl.ds(ROW, PM_ROW)], sp).wait()
            pltpu.async_copy(pbuf, ef_hbm.at[b], sp).wait()

    return k(af2, pm2)


def _tc_anchor_mask(at2, ea_row):
    def body(at_ref, ea_ref, ta_ref, eat_ref, mask_ref):
        ta_ref[:, :A_ROW] = at_ref[...]
        tail = jnp.broadcast_to(ea_ref[...][None, :], (B, EA_ROW))
        ta_ref[:, A_ROW:] = tail
        eat_ref[...] = tail
        mask_ref[...] = jnp.zeros((B, N + M), jnp.int8)

    return pl.pallas_call(
        body,
        out_shape=[
            jax.ShapeDtypeStruct((B, TA_ROW), jnp.float32),
            jax.ShapeDtypeStruct((B, EA_ROW), jnp.float32),
            jax.ShapeDtypeStruct((B, N + M), jnp.int8),
        ],
    )(at2, ea_row)


def kernel(agent_target, agent_feature, agent_mask, plan_mode_query, ego_anchor, batch_size):
    af2 = agent_feature.reshape(B, ROW)
    pm2 = plan_mode_query.reshape(B, PM_ROW)
    at2 = agent_target.reshape(B, A_ROW)
    ea_row = jnp.tile(ego_anchor.reshape(9), M)

    tif_flat, ef_flat = _sc_feature_copy(af2, pm2)
    ta_flat, eat_flat, mask_i8 = _tc_anchor_mask(at2, ea_row)

    ego_feature = ef_flat.reshape(B, M, D)
    ego_anchor_t = eat_flat.reshape(B, M, 9)
    temp_instance_feature = tif_flat.reshape(B, N + M, 1, D)
    temp_anchor = ta_flat.reshape(B, N + M, 1, 9)
    temp_mask = mask_i8.astype(jnp.bool_).reshape(B, N + M, 1)
    return (ego_feature, ego_anchor_t, temp_instance_feature, temp_anchor, temp_mask)


# speedup vs baseline: 1.4381x; 1.4381x over previous
import functools

import jax
import jax.numpy as jnp
from jax import lax
from jax.experimental import pallas as pl
from jax.experimental.pallas import tpu as pltpu
from jax.experimental.pallas import tpu_sc as plsc

B, N, D, M = 8, 2048, 256, 18

W_PER_B = 4
CHR = 128
NCH = N // (W_PER_B * CHR)


def _sc_feature_copy(af, pm):
    info = plsc.get_sparse_core_info()
    nc = info.num_cores
    mesh = plsc.VectorSubcoreMesh(core_axis_name="c", subcore_axis_name="s")

    @functools.partial(
        pl.kernel,
        mesh=mesh,
        out_type=jax.ShapeDtypeStruct((B, N + M, 1, D), jnp.float32),
        scratch_types=[
            pltpu.VMEM((CHR, D), jnp.float32),
            pltpu.VMEM((CHR, D), jnp.float32),
            pltpu.VMEM((M, D), jnp.float32),
            pltpu.SemaphoreType.DMA,
            pltpu.SemaphoreType.DMA,
            pltpu.SemaphoreType.DMA,
            pltpu.SemaphoreType.DMA,
            pltpu.SemaphoreType.DMA,
        ],
    )
    def k(af_hbm, pm_hbm, tif_hbm, buf0, buf1, pbuf, si0, si1, so0, so1, sp):
        w = lax.axis_index("s") * nc + lax.axis_index("c")
        b = w // W_PER_B
        q = w % W_PER_B
        base = q * (NCH * CHR)
        bufs = (buf0, buf1)
        sin = (si0, si1)
        sout = (so0, so1)

        cin = [None, None]
        for s in range(2):
            cin[s] = pltpu.async_copy(
                af_hbm.at[b, pl.ds(base + s * CHR, CHR), :], bufs[s], sin[s])
        for i in range(NCH):
            s = i % 2
            cin[s].wait()
            cout = pltpu.async_copy(
                bufs[s], tif_hbm.at[b, pl.ds(base + i * CHR, CHR), 0, :], sout[s])
            cout.wait()
            if i + 2 < NCH:
                cin[s] = pltpu.async_copy(
                    af_hbm.at[b, pl.ds(base + (i + 2) * CHR, CHR), :], bufs[s], sin[s])

        @pl.when(q == 0)
        def _pm():
            pltpu.async_copy(pm_hbm.at[b, 0, :, :], pbuf, sp).wait()
            pltpu.async_copy(pbuf, tif_hbm.at[b, pl.ds(N, M), 0, :], sp).wait()

    return k(af, pm)


def _tc_small(at, pm, ea):
    def body(at_ref, pm_ref, ea_ref, ta_ref, ef_ref, eat_ref, mask_ref):
        ta_ref[:, :N, 0, :] = at_ref[...]
        bc = jnp.broadcast_to(ea_ref[0][None, None, :], (B, M, 9))
        ta_ref[:, N:, 0, :] = bc
        eat_ref[...] = bc
        ef_ref[...] = pm_ref[:, 0]
        mask_ref[...] = jnp.zeros((B, N + M, 1), jnp.int8)

    return pl.pallas_call(
        body,
        out_shape=[
            jax.ShapeDtypeStruct((B, N + M, 1, 9), jnp.float32),
            jax.ShapeDtypeStruct((B, M, D), jnp.float32),
            jax.ShapeDtypeStruct((B, M, 9), jnp.float32),
            jax.ShapeDtypeStruct((B, N + M, 1), jnp.int8),
        ],
    )(at, pm, ea)


def kernel(agent_target, agent_feature, agent_mask, plan_mode_query, ego_anchor, batch_size):
    temp_instance_feature = _sc_feature_copy(agent_feature, plan_mode_query)
    temp_anchor, ego_feature, ego_anchor_t, mask_i8 = _tc_small(
        agent_target, plan_mode_query, ego_anchor)
    temp_mask = mask_i8.astype(jnp.bool_)
    return (ego_feature, ego_anchor_t, temp_instance_feature, temp_anchor, temp_mask)


# speedup vs baseline: 2.9426x; 2.0461x over previous
import functools

import jax
import jax.numpy as jnp
from jax import lax
from jax.experimental import pallas as pl
from jax.experimental.pallas import tpu as pltpu
from jax.experimental.pallas import tpu_sc as plsc

B, N, D, M = 8, 2048, 256, 18

W_PER_B = 4
CHR = 128
NCH = N // (W_PER_B * CHR)


def _sc_copy(af, pm_t, at_p, ea_bc):
    info = plsc.get_sparse_core_info()
    nc = info.num_cores
    mesh = plsc.VectorSubcoreMesh(core_axis_name="c", subcore_axis_name="s")

    @functools.partial(
        pl.kernel,
        mesh=mesh,
        out_type=[
            jax.ShapeDtypeStruct((B, N + M, 1, D), jnp.float32),
            jax.ShapeDtypeStruct((B, 9, 1, N + M), jnp.float32),
            jax.ShapeDtypeStruct((M, B, D), jnp.float32),
        ],
        scratch_types=[
            pltpu.VMEM((CHR, D), jnp.float32),
            pltpu.VMEM((CHR, D), jnp.float32),
            pltpu.VMEM((M, D), jnp.float32),
            pltpu.VMEM((9, N + M), jnp.float32),
            pltpu.VMEM((9, 128), jnp.float32),
            pltpu.SemaphoreType.DMA,
            pltpu.SemaphoreType.DMA,
            pltpu.SemaphoreType.DMA,
            pltpu.SemaphoreType.DMA,
            pltpu.SemaphoreType.DMA,
        ],
    )
    def k(af_hbm, pm_hbm, at_hbm, ea_hbm, tif_hbm, ta_hbm, ef_hbm,
          buf0, buf1, pbuf, abuf, tbuf, si0, si1, so0, so1, sq):
        w = lax.axis_index("s") * nc + lax.axis_index("c")
        b = w // W_PER_B
        q = w % W_PER_B
        base = q * (NCH * CHR)
        bufs = (buf0, buf1)
        sin = (si0, si1)
        sout = (so0, so1)

        cin = [None, None]
        for s in range(2):
            cin[s] = pltpu.async_copy(
                af_hbm.at[b, pl.ds(base + s * CHR, CHR), :], bufs[s], sin[s])
        for i in range(NCH):
            s = i % 2
            cin[s].wait()
            cout = pltpu.async_copy(
                bufs[s], tif_hbm.at[b, pl.ds(base + i * CHR, CHR), 0, :], sout[s])
            cout.wait()
            if i + 2 < NCH:
                cin[s] = pltpu.async_copy(
                    af_hbm.at[b, pl.ds(base + (i + 2) * CHR, CHR), :], bufs[s], sin[s])

        @pl.when(q == 0)
        def _pm():
            pltpu.async_copy(pm_hbm.at[b, :, 0, :], pbuf, sq).wait()
            pltpu.async_copy(pbuf, tif_hbm.at[b, pl.ds(N, M), 0, :], sq).wait()

        @pl.when(q == 1)
        def _anchor():
            pltpu.async_copy(at_hbm.at[:, b, :], abuf.at[:, pl.ds(0, N)], sq).wait()
            pltpu.async_copy(ea_hbm, tbuf, sq).wait()
            for j in range(9):
                sp = tbuf[j, pl.ds(0, 16)]
                abuf[j, pl.ds(N, 16)] = sp
                abuf[j, pl.ds(N + 2, 16)] = sp
            pltpu.async_copy(abuf, ta_hbm.at[b, :, 0, :], sq).wait()

        @pl.when(q == 2)
        def _ef():
            pltpu.async_copy(pm_hbm.at[b, :, 0, :], pbuf, sq).wait()
            pltpu.async_copy(pbuf, ef_hbm.at[:, b, :], sq).wait()

    return k(af, pm_t, at_p, ea_bc)


def _tc_small(ea):
    def body(ea_ref, mask_ref, eat_ref, bc_ref):
        col = ea_ref[...].reshape(9, 1)
        bc_ref[...] = jnp.broadcast_to(col, (9, 128))
        eat_ref[...] = jnp.broadcast_to(col[:, :, None], (9, B, M))
        mask_ref[...] = jnp.zeros((1, B, N + M), jnp.bool_)

    return pl.pallas_call(
        body,
        out_shape=[
            jax.ShapeDtypeStruct((1, B, N + M), jnp.bool_),
            jax.ShapeDtypeStruct((9, B, M), jnp.float32),
            jax.ShapeDtypeStruct((9, 128), jnp.float32),
        ],
    )(ea)


def kernel(agent_target, agent_feature, agent_mask, plan_mode_query, ego_anchor, batch_size):
    pm_t = plan_mode_query.transpose(0, 2, 1, 3)
    at_p = agent_target.transpose(2, 0, 1)

    mask_p, eat_p, ea_bc = _tc_small(ego_anchor)
    tif, ta_p, ef_p = _sc_copy(agent_feature, pm_t, at_p, ea_bc)

    ego_feature = ef_p.transpose(1, 0, 2)
    ego_anchor_t = eat_p.transpose(1, 2, 0)
    temp_anchor = ta_p.transpose(0, 3, 2, 1)
    temp_mask = mask_p.transpose(1, 2, 0)
    return (ego_feature, ego_anchor_t, tif, temp_anchor, temp_mask)
